# baseline (speedup 1.0000x reference)
import functools

import jax
import jax.numpy as jnp
from jax import lax
from jax.experimental import pallas as pl
from jax.experimental.pallas import tpu as pltpu


def _gt(av, ai, bv, bi):
    return (av > bv) | ((av == bv) & (ai < bi))


def _stage(v, i, j, desc, pos):
    vm = pltpu.roll(v, -j, 2)
    vp = pltpu.roll(v, j, 2)
    im = pltpu.roll(i, -j, 2)
    ip = pltpu.roll(i, j, 2)
    low = (pos & j) == 0
    pv = jnp.where(low, vm, vp)
    pi = jnp.where(low, im, ip)
    ga = _gt(v, i, pv, pi)
    keep = (ga == low) == desc
    return jnp.where(keep, v, pv), jnp.where(keep, i, pi)


def _topk_kernel(x_ref, vals_ref, idxs_ref, *, K, C, LOGK):
    x = x_ref[...]
    B = x.shape[0]
    v = x.reshape(B, C, K)
    ci = lax.broadcasted_iota(jnp.int32, (B, C, K), 1)
    pos = lax.broadcasted_iota(jnp.int32, (B, C, K), 2)
    i = ci * K + pos
    even = (ci & 1) == 0

    def outer(l, carry):
        v, i = carry
        kk = 1 << l
        desc = ((pos & kk) == 0) == even

        def inner(s, carry):
            v, i = carry
            j = 1 << (l - 1 - s)
            return _stage(v, i, j, desc, pos)

        return lax.fori_loop(0, l, inner, (v, i))

    v, i = lax.fori_loop(1, LOGK + 1, outer, (v, i))

    while v.shape[1] > 1:
        Ch = v.shape[1] // 2
        vr = v.reshape(B, Ch, 2, K)
        ir = i.reshape(B, Ch, 2, K)
        av, bv = vr[:, :, 0, :], vr[:, :, 1, :]
        ai, bi = ir[:, :, 0, :], ir[:, :, 1, :]
        ga = _gt(av, ai, bv, bi)
        v = jnp.where(ga, av, bv)
        i = jnp.where(ga, ai, bi)
        ci_h = lax.broadcasted_iota(jnp.int32, (B, Ch, K), 1)
        pos_h = lax.broadcasted_iota(jnp.int32, (B, Ch, K), 2)
        desc = (ci_h & 1) == 0

        def clean(s, carry):
            v, i = carry
            j = 1 << (LOGK - 1 - s)
            return _stage(v, i, j, desc, pos_h)

        v, i = lax.fori_loop(0, LOGK, clean, (v, i))

    vals_ref[...] = v[:, 0, :]
    idxs_ref[...] = i[:, 0, :]


def kernel(input_tensor, k):
    M, N = input_tensor.shape
    try:
        K = int(k)
    except jax.errors.ConcretizationTypeError:
        K = 1024
    C = N // K
    LOGK = K.bit_length() - 1
    B = min(8, M)
    grid = (M // B,)
    body = functools.partial(_topk_kernel, K=K, C=C, LOGK=LOGK)
    values, indices = pl.pallas_call(
        body,
        grid=grid,
        in_specs=[pl.BlockSpec((B, N), lambda t: (t, 0))],
        out_specs=[
            pl.BlockSpec((B, K), lambda t: (t, 0)),
            pl.BlockSpec((B, K), lambda t: (t, 0)),
        ],
        out_shape=[
            jax.ShapeDtypeStruct((M, K), jnp.float32),
            jax.ShapeDtypeStruct((M, K), jnp.int32),
        ],
    )(input_tensor)
    return (values, indices)
